# baseline (speedup 1.0000x reference)
import functools

import jax
import jax.numpy as jnp
from jax import lax
from jax.experimental import pallas as pl
from jax.experimental.pallas import tpu as pltpu
from jax.experimental.pallas import tpu_sc as plsc

HIDDEN = 1024
N = 131072
S = 8192
B = 8192
PADW = 16

NC = 2
NS = 16
NW = NC * NS
ROWS_PER_TILE = N // NW
CHUNK = 128
N_CHUNKS = ROWS_PER_TILE // CHUNK
I_PER_TILE = B // NS
I_CHUNKS = I_PER_TILE // CHUNK


def _sc_segsum_gather(gent_hbm, idx_hbm, i_hbm, zero_hbm, out_hbm,
                      tgen0, tgen1, genv, idxv, iv, gout, accum, sem, ssem):
    c = lax.axis_index("c")
    s = lax.axis_index("s")
    w = c * NS + s
    QROWS = ROWS_PER_TILE // 4
    QCH = QROWS // CHUNK

    bufs = [tgen0, tgen1]
    def stage(q, buf):
        return pltpu.async_copy(
            gent_hbm.at[:, pl.ds(w * ROWS_PER_TILE + q * QROWS, QROWS)],
            buf, ssem)
    cp = stage(0, bufs[0])
    pltpu.sync_copy(idx_hbm.at[w], idxv)
    pltpu.sync_copy(i_hbm.at[s], iv)

    zrows = S // NS
    pltpu.sync_copy(zero_hbm.at[s], accum.at[pl.ds(s * zrows, zrows)])
    plsc.subcore_barrier()

    iota16 = lax.iota(jnp.int32, 16)
    ones16 = jnp.full((16,), 1.0, jnp.float32)
    for q in range(4):
        buf = bufs[q % 2]
        cp.wait()
        if q < 3:
            cp = stage(q + 1, bufs[(q + 1) % 2])
        def chunk_body(jl, carry, buf=buf, q=q):
            j = q * QCH + jl
            def grp_body(gg, carry2):
                rv = j * CHUNK + gg * 16 + iota16
                for f in range(12):
                    vals = buf[f, pl.ds(jl * CHUNK + gg * 16, 16)]
                    plsc.store_scatter(
                        genv, [rv, jnp.full((16,), f, jnp.int32)], vals)
                plsc.store_scatter(
                    genv, [rv, jnp.full((16,), 12, jnp.int32)], ones16)
                return carry2
            lax.fori_loop(0, CHUNK // 16, grp_body, 0)
            pltpu.async_copy(genv.at[pl.ds(j * CHUNK, CHUNK)],
                             accum.at[idxv.at[j]], sem, add=True)
            return carry
        lax.fori_loop(0, QCH, chunk_body, 0)

    def drain_body(j, carry):
        pltpu.make_async_copy(genv.at[pl.ds(j * CHUNK, CHUNK)],
                              accum.at[idxv.at[j]], sem).wait()
        return carry
    lax.fori_loop(0, N_CHUNKS, drain_body, 0)
    plsc.subcore_barrier()

    def gfire_body(j, carry):
        pltpu.async_copy(accum.at[iv.at[j]],
                         gout.at[pl.ds(j * CHUNK, CHUNK)], sem)
        return carry
    lax.fori_loop(0, I_CHUNKS, gfire_body, 0)
    def gdrain_body(j, carry):
        pltpu.make_async_copy(accum.at[iv.at[j]],
                              gout.at[pl.ds(j * CHUNK, CHUNK)], sem).wait()
        return carry
    lax.fori_loop(0, I_CHUNKS, gdrain_body, 0)
    pltpu.sync_copy(gout, out_hbm.at[c, pl.ds(s * I_PER_TILE, I_PER_TILE)])


def _sc_call(gent, idx3, i3, zeros):
    mesh = plsc.VectorSubcoreMesh(core_axis_name="c", subcore_axis_name="s")
    fn = functools.partial(
        pl.kernel,
        out_type=jax.ShapeDtypeStruct((NC, B, PADW), jnp.float32),
        mesh=mesh,
        scratch_types=[
            pltpu.VMEM((12, ROWS_PER_TILE // 4), jnp.float32),
            pltpu.VMEM((12, ROWS_PER_TILE // 4), jnp.float32),
            pltpu.VMEM((ROWS_PER_TILE, PADW), jnp.float32),
            pltpu.VMEM((N_CHUNKS, CHUNK), jnp.int32),
            pltpu.VMEM((I_CHUNKS, CHUNK), jnp.int32),
            pltpu.VMEM((I_PER_TILE, PADW), jnp.float32),
            pltpu.VMEM_SHARED((S, PADW), jnp.float32),
            pltpu.SemaphoreType.DMA,
            pltpu.SemaphoreType.DMA,
        ],
        compiler_params=pltpu.CompilerParams(use_tc_tiling_on_sc=False,
                                             needs_layout_passes=False),
    )(_sc_segsum_gather)
    return fn(gent, idx3, i3, zeros)


BLK = 512


def _mlp_body(g2_ref, wemb_ref, w1_ref, b1_ref, w2_ref, b2_ref,
              gamma_ref, beta_ref):
    g = g2_ref[0] + g2_ref[1]
    cnt = g[:, 12:13]
    x = g / jnp.maximum(cnt, 1.0)
    lane = lax.broadcasted_iota(jnp.int32, (1, PADW), 1)
    x = jnp.where(lane < 13, x, 0.0)
    emb = jnp.dot(x, wemb_ref[...], preferred_element_type=jnp.float32)
    h = jnp.dot(emb, w1_ref[...], preferred_element_type=jnp.float32)
    h = h + b1_ref[...]
    h = jnp.where(h > 0, h, jnp.exp(jnp.minimum(h, 0.0)) - 1.0)
    film = jnp.dot(h, w2_ref[...], preferred_element_type=jnp.float32)
    film = film + b2_ref[...]
    gamma_ref[...] = film[:, :HIDDEN]
    beta_ref[...] = film[:, HIDDEN:]


def _mlp_call(g2, wemb16, w1, b1, w2, b2):
    nblk = B // BLK
    return pl.pallas_call(
        _mlp_body,
        grid=(nblk,),
        in_specs=[
            pl.BlockSpec((NC, BLK, PADW), lambda b: (0, b, 0)),
            pl.BlockSpec((PADW, HIDDEN), lambda b: (0, 0)),
            pl.BlockSpec((HIDDEN, HIDDEN), lambda b: (0, 0)),
            pl.BlockSpec((1, HIDDEN), lambda b: (0, 0)),
            pl.BlockSpec((HIDDEN, 2 * HIDDEN), lambda b: (0, 0)),
            pl.BlockSpec((1, 2 * HIDDEN), lambda b: (0, 0)),
        ],
        out_specs=[
            pl.BlockSpec((BLK, HIDDEN), lambda b: (b, 0)),
            pl.BlockSpec((BLK, HIDDEN), lambda b: (b, 0)),
        ],
        out_shape=[
            jax.ShapeDtypeStruct((B, HIDDEN), jnp.float32),
            jax.ShapeDtypeStruct((B, HIDDEN), jnp.float32),
        ],
    )(g2, wemb16, w1, b1, w2, b2)


def kernel(layer_idx, gen, gen_idx, i, W_emb, W1, b1, W2, b2):
    del layer_idx
    gen = gen.astype(jnp.float32)
    idx = gen_idx.astype(jnp.int32)
    ii = i.astype(jnp.int32)

    gent = gen.T
    idx3 = idx.reshape(NW, N_CHUNKS, CHUNK)
    i3 = ii.reshape(NS, I_CHUNKS, CHUNK)
    zeros = jnp.zeros((NS, S // NS, PADW), jnp.float32)

    g2 = _sc_call(gent, idx3, i3, zeros)

    wemb16 = jnp.pad(W_emb.astype(jnp.float32), ((0, PADW - 12), (0, 0)))
    gamma, beta = _mlp_call(g2, wemb16,
                            W1.astype(jnp.float32),
                            b1.astype(jnp.float32).reshape(1, HIDDEN),
                            W2.astype(jnp.float32),
                            b2.astype(jnp.float32).reshape(1, 2 * HIDDEN))
    return (gamma, beta)


# speedup vs baseline: 1.0063x; 1.0063x over previous
import functools

import jax
import jax.numpy as jnp
from jax import lax
from jax.experimental import pallas as pl
from jax.experimental.pallas import tpu as pltpu
from jax.experimental.pallas import tpu_sc as plsc

HIDDEN = 1024
N = 131072
S = 8192
B = 8192
PADW = 16

NC = 2
NS = 16
NW = NC * NS
ROWS_PER_TILE = N // NW
CHUNK = 128
N_CHUNKS = ROWS_PER_TILE // CHUNK
I_PER_TILE = B // NS
I_CHUNKS = I_PER_TILE // CHUNK


def _sc_segsum_gather(gent_hbm, idx_hbm, i_hbm, zero_hbm, out_hbm,
                      tgen0, tgen1, genv, idxv, iv, gout, accum, sem, ssem):
    c = lax.axis_index("c")
    s = lax.axis_index("s")
    w = c * NS + s
    QROWS = ROWS_PER_TILE // 4
    QCH = QROWS // CHUNK

    bufs = [tgen0, tgen1]
    def stage(q, buf):
        return pltpu.async_copy(
            gent_hbm.at[:, pl.ds(w * ROWS_PER_TILE + q * QROWS, QROWS)],
            buf, ssem)
    cp = stage(0, bufs[0])
    pltpu.sync_copy(idx_hbm.at[w], idxv)
    pltpu.sync_copy(i_hbm.at[s], iv)

    zrows = S // NS
    pltpu.sync_copy(zero_hbm.at[s], accum.at[pl.ds(s * zrows, zrows)])
    plsc.subcore_barrier()

    iota16 = lax.iota(jnp.int32, 16)
    ones16 = jnp.full((16,), 1.0, jnp.float32)
    for q in range(4):
        buf = bufs[q % 2]
        cp.wait()
        if q < 3:
            cp = stage(q + 1, bufs[(q + 1) % 2])
        def chunk_body(jl, carry, buf=buf, q=q):
            j = q * QCH + jl
            def grp_body(gg, carry2):
                rv = j * CHUNK + gg * 16 + iota16
                for f in range(12):
                    vals = buf[f, pl.ds(jl * CHUNK + gg * 16, 16)]
                    plsc.store_scatter(
                        genv, [rv, jnp.full((16,), f, jnp.int32)], vals)
                plsc.store_scatter(
                    genv, [rv, jnp.full((16,), 12, jnp.int32)], ones16)
                return carry2
            lax.fori_loop(0, CHUNK // 16, grp_body, 0)
            pltpu.async_copy(genv.at[pl.ds(j * CHUNK, CHUNK)],
                             accum.at[idxv.at[j]], sem, add=True)
            return carry
        lax.fori_loop(0, QCH, chunk_body, 0)

    def drain_body(j, carry):
        pltpu.make_async_copy(genv.at[pl.ds(j * CHUNK, CHUNK)],
                              accum.at[idxv.at[j]], sem).wait()
        return carry
    lax.fori_loop(0, N_CHUNKS, drain_body, 0)
    plsc.subcore_barrier()

    def gfire_body(j, carry):
        pltpu.async_copy(accum.at[iv.at[j]],
                         gout.at[pl.ds(j * CHUNK, CHUNK)], sem)
        return carry
    lax.fori_loop(0, I_CHUNKS, gfire_body, 0)
    def gdrain_body(j, carry):
        pltpu.make_async_copy(accum.at[iv.at[j]],
                              gout.at[pl.ds(j * CHUNK, CHUNK)], sem).wait()
        return carry
    lax.fori_loop(0, I_CHUNKS, gdrain_body, 0)
    pltpu.sync_copy(gout, out_hbm.at[c, pl.ds(s * I_PER_TILE, I_PER_TILE)])


def _sc_call(gent, idx3, i3, zeros):
    mesh = plsc.VectorSubcoreMesh(core_axis_name="c", subcore_axis_name="s")
    fn = functools.partial(
        pl.kernel,
        out_type=jax.ShapeDtypeStruct((NC, B, PADW), jnp.float32),
        mesh=mesh,
        scratch_types=[
            pltpu.VMEM((12, ROWS_PER_TILE // 4), jnp.float32),
            pltpu.VMEM((12, ROWS_PER_TILE // 4), jnp.float32),
            pltpu.VMEM((ROWS_PER_TILE, PADW), jnp.float32),
            pltpu.VMEM((N_CHUNKS, CHUNK), jnp.int32),
            pltpu.VMEM((I_CHUNKS, CHUNK), jnp.int32),
            pltpu.VMEM((I_PER_TILE, PADW), jnp.float32),
            pltpu.VMEM_SHARED((S, PADW), jnp.float32),
            pltpu.SemaphoreType.DMA,
            pltpu.SemaphoreType.DMA,
        ],
        compiler_params=pltpu.CompilerParams(use_tc_tiling_on_sc=False,
                                             needs_layout_passes=False),
    )(_sc_segsum_gather)
    return fn(gent, idx3, i3, zeros)


BLK = 1024


def _mlp_body(g2_ref, wemb_ref, w1_ref, b1_ref, w2_ref, b2_ref,
              gamma_ref, beta_ref):
    g = g2_ref[0] + g2_ref[1]
    cnt = g[:, 12:13]
    x = g / jnp.maximum(cnt, 1.0)
    lane = lax.broadcasted_iota(jnp.int32, (1, PADW), 1)
    x = jnp.where(lane < 13, x, 0.0)
    emb = jnp.dot(x, wemb_ref[...], preferred_element_type=jnp.float32)
    h = jnp.dot(emb, w1_ref[...], preferred_element_type=jnp.float32)
    h = h + b1_ref[...]
    h = jnp.where(h > 0, h, jnp.exp(jnp.minimum(h, 0.0)) - 1.0)
    gamma_ref[...] = jnp.dot(h, w2_ref[:, :HIDDEN],
                             preferred_element_type=jnp.float32) \
        + b2_ref[:, :HIDDEN]
    beta_ref[...] = jnp.dot(h, w2_ref[:, HIDDEN:],
                            preferred_element_type=jnp.float32) \
        + b2_ref[:, HIDDEN:]


def _mlp_call(g2, wemb16, w1, b1, w2, b2):
    nblk = B // BLK
    return pl.pallas_call(
        _mlp_body,
        grid=(nblk,),
        in_specs=[
            pl.BlockSpec((NC, BLK, PADW), lambda b: (0, b, 0)),
            pl.BlockSpec((PADW, HIDDEN), lambda b: (0, 0)),
            pl.BlockSpec((HIDDEN, HIDDEN), lambda b: (0, 0)),
            pl.BlockSpec((1, HIDDEN), lambda b: (0, 0)),
            pl.BlockSpec((HIDDEN, 2 * HIDDEN), lambda b: (0, 0)),
            pl.BlockSpec((1, 2 * HIDDEN), lambda b: (0, 0)),
        ],
        out_specs=[
            pl.BlockSpec((BLK, HIDDEN), lambda b: (b, 0)),
            pl.BlockSpec((BLK, HIDDEN), lambda b: (b, 0)),
        ],
        out_shape=[
            jax.ShapeDtypeStruct((B, HIDDEN), jnp.float32),
            jax.ShapeDtypeStruct((B, HIDDEN), jnp.float32),
        ],
    )(g2, wemb16, w1, b1, w2, b2)


def kernel(layer_idx, gen, gen_idx, i, W_emb, W1, b1, W2, b2):
    del layer_idx
    gen = gen.astype(jnp.float32)
    idx = gen_idx.astype(jnp.int32)
    ii = i.astype(jnp.int32)

    gent = gen.T
    idx3 = idx.reshape(NW, N_CHUNKS, CHUNK)
    i3 = ii.reshape(NS, I_CHUNKS, CHUNK)
    zeros = jnp.zeros((NS, S // NS, PADW), jnp.float32)

    g2 = _sc_call(gent, idx3, i3, zeros)

    wemb16 = jnp.pad(W_emb.astype(jnp.float32), ((0, PADW - 12), (0, 0)))
    gamma, beta = _mlp_call(g2, wemb16,
                            W1.astype(jnp.float32),
                            b1.astype(jnp.float32).reshape(1, HIDDEN),
                            W2.astype(jnp.float32),
                            b2.astype(jnp.float32).reshape(1, 2 * HIDDEN))
    return (gamma, beta)
